# baseline (speedup 1.0000x reference)
import jax
import jax.numpy as jnp
from jax.experimental import pallas as pl
from jax.experimental.pallas import tpu as pltpu

_NCLS = 512
_NB = 4


def _glvq_body(x_ref, p_ref, o_ref):
    x = x_ref[:]
    p = p_ref[:]
    xx = jnp.sum(x * x, axis=1, keepdims=True)
    pp = jnp.sum(p * p, axis=1, keepdims=True)
    ones_x = jnp.ones_like(xx)
    ones_p = jnp.ones_like(pp)
    xa = jnp.concatenate([x * -2.0, xx, ones_x], axis=1)
    pa = jnp.concatenate([p, ones_p, pp], axis=1)
    dn = (((1,), (1,)), ((), ()))
    d2 = jax.lax.dot_general(xa, pa, dn, preferred_element_type=jnp.float32)
    na = d2[:, :_NCLS]
    nb = d2[:, _NCLS:]
    m = jnp.where(na < nb, na, nb)
    ab = jnp.abs(m) + 1e-30
    o_ref[:] = ab * jax.lax.rsqrt(ab)


def kernel(x, protos):
    batch, d = x.shape
    bb = batch // _NB
    return pl.pallas_call(
        _glvq_body,
        grid=(_NB,),
        in_specs=[
            pl.BlockSpec((bb, d), lambda i: (i, 0)),
            pl.BlockSpec(protos.shape, lambda i: (0, 0)),
        ],
        out_specs=pl.BlockSpec((bb, _NCLS), lambda i: (i, 0)),
        out_shape=jax.ShapeDtypeStruct((batch, _NCLS), jnp.float32),
    )(x, protos)


# speedup vs baseline: 1.1624x; 1.1624x over previous
import jax
import jax.numpy as jnp
from jax.experimental import pallas as pl
from jax.experimental.pallas import tpu as pltpu

_NCLS = 512
_NB = 2


def _glvq_body(x_ref, p_ref, o_ref, pa_ref):
    @pl.when(pl.program_id(0) == 0)
    def _build_protos():
        p = p_ref[:]
        pp = jnp.sum(p * p, axis=1, keepdims=True)
        pa_ref[:] = jnp.concatenate([p, jnp.ones_like(pp), pp], axis=1)

    x = x_ref[:]
    xx = jnp.sum(x * x, axis=1, keepdims=True)
    xa = jnp.concatenate([x * -2.0, xx, jnp.ones_like(xx)], axis=1)
    dn = (((1,), (1,)), ((), ()))
    d2 = jax.lax.dot_general(xa, pa_ref[:], dn,
                             preferred_element_type=jnp.float32)
    na = d2[:, :_NCLS]
    nb = d2[:, _NCLS:]
    m = jnp.where(na < nb, na, nb)
    ab = jnp.abs(m) + 1e-30
    o_ref[:] = ab * jax.lax.rsqrt(ab)


def kernel(x, protos):
    batch, d = x.shape
    bb = batch // _NB
    return pl.pallas_call(
        _glvq_body,
        grid=(_NB,),
        in_specs=[
            pl.BlockSpec((bb, d), lambda i: (i, 0)),
            pl.BlockSpec(protos.shape, lambda i: (0, 0)),
        ],
        out_specs=pl.BlockSpec((bb, _NCLS), lambda i: (i, 0)),
        out_shape=jax.ShapeDtypeStruct((batch, _NCLS), jnp.float32),
        scratch_shapes=[pltpu.VMEM((2 * _NCLS, d + 2), jnp.float32)],
    )(x, protos)


# speedup vs baseline: 1.1752x; 1.0110x over previous
import jax
import jax.numpy as jnp
from jax.experimental import pallas as pl
from jax.experimental.pallas import tpu as pltpu

_NCLS = 512
_NB = 2


def _glvq_body(x_ref, p_ref, o_ref):
    x = x_ref[:]
    p = p_ref[:]
    xx = jnp.sum(x * x, axis=1, keepdims=True)
    pp = jnp.sum(p * p, axis=1, keepdims=True)
    xa = jnp.concatenate([x * -2.0, xx, jnp.ones_like(xx)], axis=1)
    pa = jnp.concatenate([p, jnp.ones_like(pp), pp], axis=1)
    dn = (((1,), (1,)), ((), ()))
    d2 = jax.lax.dot_general(xa.astype(jnp.bfloat16), pa.astype(jnp.bfloat16),
                             dn, preferred_element_type=jnp.float32)
    na = d2[:, :_NCLS]
    nb = d2[:, _NCLS:]
    m = jnp.where(na < nb, na, nb)
    ab = jnp.abs(m) + 1e-30
    o_ref[:] = ab * jax.lax.rsqrt(ab)


def kernel(x, protos):
    batch, d = x.shape
    bb = batch // _NB
    return pl.pallas_call(
        _glvq_body,
        grid=(_NB,),
        in_specs=[
            pl.BlockSpec((bb, d), lambda i: (i, 0)),
            pl.BlockSpec(protos.shape, lambda i: (0, 0)),
        ],
        out_specs=pl.BlockSpec((bb, _NCLS), lambda i: (i, 0)),
        out_shape=jax.ShapeDtypeStruct((batch, _NCLS), jnp.float32),
    )(x, protos)


# speedup vs baseline: 1.2100x; 1.0296x over previous
import jax
import jax.numpy as jnp
from jax.experimental import pallas as pl
from jax.experimental.pallas import tpu as pltpu

_NCLS = 512


def _half(x, pa, o_buf, rows):
    xx = jnp.sum(x * x, axis=1, keepdims=True)
    xa = jnp.concatenate([x * -2.0, xx, jnp.ones_like(xx)], axis=1)
    dn = (((1,), (1,)), ((), ()))
    d2 = jax.lax.dot_general(xa, pa, dn, preferred_element_type=jnp.float32)
    m = jnp.where(d2[:, :_NCLS] < d2[:, _NCLS:], d2[:, :_NCLS], d2[:, _NCLS:])
    ab = jnp.abs(m) + 1e-30
    o_buf[rows, :] = ab * jax.lax.rsqrt(ab)


def _glvq_body(x_ref, p_ref, o_hbm, buf, sem0, sem1):
    p = p_ref[:]
    pp = jnp.sum(p * p, axis=1, keepdims=True)
    pa = jnp.concatenate([p, jnp.ones_like(pp), pp], axis=1)
    half = x_ref.shape[0] // 2

    r0 = pl.ds(0, half)
    _half(x_ref[r0, :], pa, buf, r0)
    cp0 = pltpu.make_async_copy(buf.at[r0, :], o_hbm.at[r0, :], sem0)
    cp0.start()

    r1 = pl.ds(half, half)
    _half(x_ref[r1, :], pa, buf, r1)
    cp1 = pltpu.make_async_copy(buf.at[r1, :], o_hbm.at[r1, :], sem1)
    cp1.start()

    cp0.wait()
    cp1.wait()


def kernel(x, protos):
    batch = x.shape[0]
    return pl.pallas_call(
        _glvq_body,
        out_shape=jax.ShapeDtypeStruct((batch, _NCLS), jnp.float32),
        out_specs=pl.BlockSpec(memory_space=pltpu.MemorySpace.HBM),
        scratch_shapes=[
            pltpu.VMEM((batch, _NCLS), jnp.float32),
            pltpu.SemaphoreType.DMA,
            pltpu.SemaphoreType.DMA,
        ],
    )(x, protos)


# speedup vs baseline: 1.2458x; 1.0296x over previous
import jax
import jax.numpy as jnp
from jax.experimental import pallas as pl
from jax.experimental.pallas import tpu as pltpu

_NCLS = 512
_NCHUNK = 4


def _chunk(x, pa, o_buf, rows):
    xx = jnp.sum(x * x, axis=1, keepdims=True)
    xa = jnp.concatenate([x * -2.0, xx, jnp.ones_like(xx)], axis=1)
    dn = (((1,), (1,)), ((), ()))
    d2 = jax.lax.dot_general(xa, pa, dn, preferred_element_type=jnp.float32)
    m = jnp.where(d2[:, :_NCLS] < d2[:, _NCLS:], d2[:, :_NCLS], d2[:, _NCLS:])
    ab = jnp.abs(m) + 1e-30
    o_buf[rows, :] = ab * jax.lax.rsqrt(ab)


def _glvq_body(x_ref, p_ref, o_hbm, buf, *sems):
    p = p_ref[:]
    pp = jnp.sum(p * p, axis=1, keepdims=True)
    pa = jnp.concatenate([p, jnp.ones_like(pp), pp], axis=1)
    chunk = x_ref.shape[0] // _NCHUNK

    copies = []
    for i in range(_NCHUNK):
        rows = pl.ds(i * chunk, chunk)
        _chunk(x_ref[rows, :], pa, buf, rows)
        cp = pltpu.make_async_copy(buf.at[rows, :], o_hbm.at[rows, :], sems[i])
        cp.start()
        copies.append(cp)
    for cp in copies:
        cp.wait()


def kernel(x, protos):
    batch = x.shape[0]
    return pl.pallas_call(
        _glvq_body,
        out_shape=jax.ShapeDtypeStruct((batch, _NCLS), jnp.float32),
        out_specs=pl.BlockSpec(memory_space=pltpu.MemorySpace.HBM),
        scratch_shapes=[pltpu.VMEM((batch, _NCLS), jnp.float32)]
        + [pltpu.SemaphoreType.DMA] * _NCHUNK,
    )(x, protos)


# speedup vs baseline: 1.2617x; 1.0127x over previous
import jax
import jax.numpy as jnp
from jax.experimental import pallas as pl
from jax.experimental.pallas import tpu as pltpu

_NCLS = 512
_NCHUNK = 8


def _chunk(x, pa, o_buf, rows):
    xx = jnp.sum(x * x, axis=1, keepdims=True)
    xa = jnp.concatenate([x * -2.0, xx, jnp.ones_like(xx)], axis=1)
    dn = (((1,), (1,)), ((), ()))
    d2 = jax.lax.dot_general(xa, pa, dn, preferred_element_type=jnp.float32)
    m = jnp.where(d2[:, :_NCLS] < d2[:, _NCLS:], d2[:, :_NCLS], d2[:, _NCLS:])
    ab = jnp.abs(m) + 1e-30
    o_buf[rows, :] = ab * jax.lax.rsqrt(ab)


def _glvq_body(x_ref, p_ref, o_hbm, buf, *sems):
    p = p_ref[:]
    pp = jnp.sum(p * p, axis=1, keepdims=True)
    pa = jnp.concatenate([p, jnp.ones_like(pp), pp], axis=1)
    chunk = x_ref.shape[0] // _NCHUNK

    copies = []
    for i in range(_NCHUNK):
        rows = pl.ds(i * chunk, chunk)
        _chunk(x_ref[rows, :], pa, buf, rows)
        cp = pltpu.make_async_copy(buf.at[rows, :], o_hbm.at[rows, :], sems[i])
        cp.start()
        copies.append(cp)
    for cp in copies:
        cp.wait()


def kernel(x, protos):
    batch = x.shape[0]
    return pl.pallas_call(
        _glvq_body,
        out_shape=jax.ShapeDtypeStruct((batch, _NCLS), jnp.float32),
        out_specs=pl.BlockSpec(memory_space=pltpu.MemorySpace.HBM),
        scratch_shapes=[pltpu.VMEM((batch, _NCLS), jnp.float32)]
        + [pltpu.SemaphoreType.DMA] * _NCHUNK,
    )(x, protos)
